# Initial kernel scaffold; baseline (speedup 1.0000x reference)
import jax
import jax.numpy as jnp
from jax.experimental import pallas as pl


def kernel(x, edge_index, edge_attr, W_in, b_in, Wl0, bl0, Wr0, g0, be0, rm0, rv0, Wl1, bl1, Wr1, g1, be1, rm1, rv1, Wc1, bc1, Wc2, bc2):
    raise NotImplementedError("write your pallas kernel here")


# speedup vs baseline: 6.0126x; 6.0126x over previous
import functools

import jax
import jax.numpy as jnp
from jax import lax
from jax.experimental import pallas as pl
from jax.experimental.pallas import tpu as pltpu
from jax.experimental.pallas import tpu_sc as plsc

N = 10000
H = 128
E = 320000

NC = 2
NS = 16
NW = NC * NS
CHUNK = 64
CPW = 160
IPH = 16
E_PAD = NW * CPW * CHUNK
R_PAD = 10240
RPW = R_PAD // NS

BLK = 1000


def _fill_rows(rows_v, val):
    v = jnp.full((16,), val, jnp.float32)

    def frow(i, carry):
        for k in range(H // 16):
            rows_v[i, pl.ds(k * 16, 16)] = v
        return carry

    lax.fori_loop(0, CHUNK, frow, 0)


def _zero_acc(rows_v, agg_s, rowbase):
    _fill_rows(rows_v, 0.0)
    for k in range(RPW // CHUNK):
        pltpu.sync_copy(rows_v, agg_s.at[pl.ds(rowbase + k * CHUNK, CHUNK)])


def _flush_acc(rows_v, agg_s, out_hbm, cid, rowbase):
    for k in range(RPW // CHUNK):
        r = rowbase + k * CHUNK
        pltpu.sync_copy(agg_s.at[pl.ds(r, CHUNK)], rows_v)
        pltpu.sync_copy(rows_v, out_hbm.at[cid, pl.ds(r, CHUNK)])


def _sc_agg_body(with_counts, *refs):
    if with_counts:
        (h_hbm, src_hbm, dst_hbm,
         p_hbm, cnt_hbm,
         src_v, dst_v, rows_v, agg_s, sem) = refs
    else:
        (h_hbm, src_hbm, dst_hbm,
         p_hbm,
         src_v, dst_v, rows_v, agg_s, sem) = refs

    cid = lax.axis_index("c")
    sid = lax.axis_index("s")
    wid = sid * NC + cid
    rowbase = sid * RPW

    if with_counts:
        _zero_acc(rows_v, agg_s, rowbase)
        plsc.subcore_barrier()
        _fill_rows(rows_v, 1.0)

        def cstep(j, carry):
            pltpu.sync_copy(rows_v, agg_s.at[dst_v.at[j]], add=True)
            return carry

        for phase in range(CPW // IPH):
            pltpu.sync_copy(dst_hbm.at[wid, pl.ds(phase * IPH, IPH)], dst_v)
            lax.fori_loop(0, IPH, cstep, 0)
        plsc.subcore_barrier()
        _flush_acc(rows_v, agg_s, cnt_hbm, cid, rowbase)
        plsc.subcore_barrier()

    _zero_acc(rows_v, agg_s, rowbase)
    plsc.subcore_barrier()

    def step(j, carry):
        pltpu.async_copy(h_hbm.at[src_v.at[j]], rows_v, sem).wait()
        pltpu.sync_copy(rows_v, agg_s.at[dst_v.at[j]], add=True)
        return carry

    for phase in range(CPW // IPH):
        pltpu.sync_copy(src_hbm.at[wid, pl.ds(phase * IPH, IPH)], src_v)
        pltpu.sync_copy(dst_hbm.at[wid, pl.ds(phase * IPH, IPH)], dst_v)
        lax.fori_loop(0, IPH, step, 0)
    plsc.subcore_barrier()
    _flush_acc(rows_v, agg_s, p_hbm, cid, rowbase)


def _make_sc_agg(with_counts):
    mesh = plsc.VectorSubcoreMesh(core_axis_name="c", subcore_axis_name="s")
    out_type = [jax.ShapeDtypeStruct((NC, R_PAD, H), jnp.float32)]
    scratch = [
        pltpu.VMEM((IPH, CHUNK), jnp.int32),
        pltpu.VMEM((IPH, CHUNK), jnp.int32),
        pltpu.VMEM((CHUNK, H), jnp.float32),
    ]
    if with_counts:
        out_type.append(jax.ShapeDtypeStruct((NC, R_PAD, H), jnp.float32))
    scratch.append(pltpu.VMEM_SHARED((R_PAD, H), jnp.float32))
    scratch.append(pltpu.SemaphoreType.DMA)
    return pl.kernel(
        functools.partial(_sc_agg_body, with_counts),
        out_type=tuple(out_type),
        mesh=mesh,
        scratch_types=scratch,
        name="sc_segment_sum" + ("_cnt" if with_counts else ""),
    )


def _tc_in_body(x_ref, w_ref, b_ref, o_ref):
    o_ref[...] = jax.nn.relu(
        jnp.dot(x_ref[...], w_ref[...], preferred_element_type=jnp.float32)
        + b_ref[...])


def _tc_layer_body(p_ref, c_ref, h_ref, wl_ref, wr_ref, s_ref, t_ref, o_ref):
    cnt = jnp.maximum(c_ref[0, :, :1] + c_ref[1, :, :1], 1.0)
    agg = (p_ref[0] + p_ref[1]) / cnt
    h = h_ref[...]
    z = (jnp.dot(agg, wl_ref[...], preferred_element_type=jnp.float32)
         + jnp.dot(h, wr_ref[...], preferred_element_type=jnp.float32))
    o_ref[...] = h + jax.nn.relu(z * s_ref[...] + t_ref[...])


def _tc_layer_head_body(p_ref, c_ref, h_ref, wl_ref, wr_ref, s_ref, t_ref,
                        wc1_ref, bc1_ref, wc2_ref, bc2_ref, o_ref):
    cnt = jnp.maximum(c_ref[0, :, :1] + c_ref[1, :, :1], 1.0)
    agg = (p_ref[0] + p_ref[1]) / cnt
    h = h_ref[...]
    z = (jnp.dot(agg, wl_ref[...], preferred_element_type=jnp.float32)
         + jnp.dot(h, wr_ref[...], preferred_element_type=jnp.float32))
    h2 = h + jax.nn.relu(z * s_ref[...] + t_ref[...])
    y = jax.nn.relu(
        jnp.dot(h2, wc1_ref[...], preferred_element_type=jnp.float32)
        + bc1_ref[...])
    o_ref[...] = (jnp.dot(y, wc2_ref[...], preferred_element_type=jnp.float32)
                  + bc2_ref[...])


def _row(shape):
    return pl.BlockSpec(shape, lambda i: (0,) * len(shape))


def _tc_in(x, w, b):
    return pl.pallas_call(
        _tc_in_body,
        grid=(N // BLK,),
        in_specs=[
            pl.BlockSpec((BLK, H), lambda i: (i, 0)),
            _row((H, H)),
            _row((1, H)),
        ],
        out_specs=pl.BlockSpec((BLK, H), lambda i: (i, 0)),
        out_shape=jax.ShapeDtypeStruct((N, H), jnp.float32),
    )(x, w, b)


def _tc_layer(p, c, h, wl, wr, s, t):
    return pl.pallas_call(
        _tc_layer_body,
        grid=(N // BLK,),
        in_specs=[
            pl.BlockSpec((NC, BLK, H), lambda i: (0, i, 0)),
            pl.BlockSpec((NC, BLK, H), lambda i: (0, i, 0)),
            pl.BlockSpec((BLK, H), lambda i: (i, 0)),
            _row((H, H)), _row((H, H)), _row((1, H)), _row((1, H)),
        ],
        out_specs=pl.BlockSpec((BLK, H), lambda i: (i, 0)),
        out_shape=jax.ShapeDtypeStruct((N, H), jnp.float32),
    )(p, c, h, wl, wr, s, t)


def _tc_layer_head(p, c, h, wl, wr, s, t, wc1, bc1, wc2, bc2):
    return pl.pallas_call(
        _tc_layer_head_body,
        grid=(N // BLK,),
        in_specs=[
            pl.BlockSpec((NC, BLK, H), lambda i: (0, i, 0)),
            pl.BlockSpec((NC, BLK, H), lambda i: (0, i, 0)),
            pl.BlockSpec((BLK, H), lambda i: (i, 0)),
            _row((H, H)), _row((H, H)), _row((1, H)), _row((1, H)),
            _row((H, H // 2)), _row((1, H // 2)),
            _row((H // 2, 1)), _row((1, 1)),
        ],
        out_specs=pl.BlockSpec((BLK, 1), lambda i: (i, 0)),
        out_shape=jax.ShapeDtypeStruct((N, 1), jnp.float32),
    )(p, c, h, wl, wr, s, t, wc1, bc1, wc2, bc2)


def _fold_bn(g, be, rm, rv, bl, eps=1e-5):
    s = g / jnp.sqrt(rv + eps)
    t = be - rm * s + bl * s
    return s.reshape(1, H), t.reshape(1, H)


def kernel(x, edge_index, edge_attr, W_in, b_in, Wl0, bl0, Wr0, g0, be0, rm0,
           rv0, Wl1, bl1, Wr1, g1, be1, rm1, rv1, Wc1, bc1, Wc2, bc2):
    src = edge_index[0]
    dst = edge_index[1]
    npad = E_PAD - E
    pad_src = (jnp.arange(npad, dtype=jnp.int32) * 131) % N
    pad_dst = N + (jnp.arange(npad, dtype=jnp.int32) % (R_PAD - N))
    src_r = jnp.concatenate([src, pad_src]).reshape(NW, CPW, CHUNK)
    dst_r = jnp.concatenate([dst, pad_dst]).reshape(NW, CPW, CHUNK)

    s0, t0 = _fold_bn(g0, be0, rm0, rv0, bl0)
    s1, t1 = _fold_bn(g1, be1, rm1, rv1, bl1)

    h0 = _tc_in(x, W_in, b_in.reshape(1, H))
    p0, cnt = _make_sc_agg(True)(h0, src_r, dst_r)
    h1 = _tc_layer(p0, cnt, h0, Wl0, Wr0, s0, t0)
    (p1,) = _make_sc_agg(False)(h1, src_r, dst_r)
    return _tc_layer_head(p1, cnt, h1, Wl1, Wr1, s1, t1,
                          Wc1, bc1.reshape(1, H // 2), Wc2, bc2.reshape(1, 1))


# speedup vs baseline: 9.2721x; 1.5421x over previous
import functools

import jax
import jax.numpy as jnp
from jax import lax
from jax.experimental import pallas as pl
from jax.experimental.pallas import tpu as pltpu
from jax.experimental.pallas import tpu_sc as plsc

N = 10000
H = 128
E = 320000

NC = 2
NS = 16
NW = NC * NS
CHUNK = 64
CPW = 160
NPH = 2
PPC = CPW // NPH
E_PAD = NW * CPW * CHUNK
R_PAD = 10240
RPW = R_PAD // NS

BLK = 1000


def _fill_rows(rows_v, val):
    v = jnp.full((16,), val, jnp.float32)

    def frow(i, carry):
        for k in range(H // 16):
            rows_v[i, pl.ds(k * 16, 16)] = v
        return carry

    lax.fori_loop(0, CHUNK, frow, 0)


def _zero_acc(rows_v, agg_s, rowbase):
    _fill_rows(rows_v, 0.0)
    for k in range(RPW // CHUNK):
        pltpu.sync_copy(rows_v, agg_s.at[pl.ds(rowbase + k * CHUNK, CHUNK)])


def _flush_acc(rows_v, agg_s, out_hbm, cid, rowbase):
    for k in range(RPW // CHUNK):
        r = rowbase + k * CHUNK
        pltpu.sync_copy(agg_s.at[pl.ds(r, CHUNK)], rows_v)
        pltpu.sync_copy(rows_v, out_hbm.at[cid, pl.ds(r, CHUNK)])


def _sc_agg_body(with_counts, *refs):
    if with_counts:
        (h_hbm, src_hbm, dst_hbm,
         p_hbm, cnt_hbm,
         src_v, dst_v, rows0, rows1, agg_s, sem0, sem1) = refs
    else:
        (h_hbm, src_hbm, dst_hbm,
         p_hbm,
         src_v, dst_v, rows0, rows1, agg_s, sem0, sem1) = refs

    cid = lax.axis_index("c")
    sid = lax.axis_index("s")
    wid = sid * NC + cid
    rowbase = sid * RPW

    if with_counts:
        _zero_acc(rows0, agg_s, rowbase)
        plsc.subcore_barrier()
        _fill_rows(rows0, 1.0)

        def cstep(j, carry):
            pltpu.sync_copy(rows0, agg_s.at[dst_v.at[j]], add=True)
            return carry

        for ph in range(NPH):
            pltpu.sync_copy(dst_hbm.at[wid, pl.ds(ph * PPC, PPC)], dst_v)
            lax.fori_loop(0, PPC, cstep, 0)
        plsc.subcore_barrier()
        _flush_acc(rows0, agg_s, cnt_hbm, cid, rowbase)
        plsc.subcore_barrier()

    _zero_acc(rows0, agg_s, rowbase)
    plsc.subcore_barrier()

    bufs = (rows0, rows1)
    sems = (sem0, sem1)

    def pair(g, carry):
        for b in range(2):
            j = 2 * g + b
            pltpu.make_async_copy(h_hbm.at[src_v.at[0]], bufs[b],
                                  sems[b]).wait()
            pltpu.sync_copy(bufs[b], agg_s.at[dst_v.at[j]], add=True)
            pltpu.async_copy(h_hbm.at[src_v.at[j + 2]], bufs[b], sems[b])
        return carry

    for ph in range(NPH):
        pltpu.sync_copy(src_hbm.at[wid, pl.ds(ph * PPC, PPC)], src_v)
        pltpu.sync_copy(dst_hbm.at[wid, pl.ds(ph * PPC, PPC)], dst_v)
        pltpu.async_copy(h_hbm.at[src_v.at[0]], rows0, sem0)
        pltpu.async_copy(h_hbm.at[src_v.at[1]], rows1, sem1)
        lax.fori_loop(0, PPC // 2 - 1, pair, 0)
        for b in range(2):
            pltpu.make_async_copy(h_hbm.at[src_v.at[0]], bufs[b],
                                  sems[b]).wait()
            pltpu.sync_copy(bufs[b], agg_s.at[dst_v.at[PPC - 2 + b]],
                            add=True)

    plsc.subcore_barrier()
    _flush_acc(rows0, agg_s, p_hbm, cid, rowbase)


def _make_sc_agg(with_counts):
    mesh = plsc.VectorSubcoreMesh(core_axis_name="c", subcore_axis_name="s")
    out_type = [jax.ShapeDtypeStruct((NC, R_PAD, H), jnp.float32)]
    scratch = [
        pltpu.VMEM((PPC, CHUNK), jnp.int32),
        pltpu.VMEM((PPC, CHUNK), jnp.int32),
        pltpu.VMEM((CHUNK, H), jnp.float32),
        pltpu.VMEM((CHUNK, H), jnp.float32),
    ]
    if with_counts:
        out_type.append(jax.ShapeDtypeStruct((NC, R_PAD, H), jnp.float32))
    scratch.append(pltpu.VMEM_SHARED((R_PAD, H), jnp.float32))
    scratch.append(pltpu.SemaphoreType.DMA)
    scratch.append(pltpu.SemaphoreType.DMA)
    return pl.kernel(
        functools.partial(_sc_agg_body, with_counts),
        out_type=tuple(out_type),
        mesh=mesh,
        scratch_types=scratch,
        name="sc_segment_sum" + ("_cnt" if with_counts else ""),
    )


def _tc_in_body(x_ref, w_ref, b_ref, o_ref):
    o_ref[...] = jax.nn.relu(
        jnp.dot(x_ref[...], w_ref[...], preferred_element_type=jnp.float32)
        + b_ref[...])


def _tc_layer_body(p_ref, c_ref, h_ref, wl_ref, wr_ref, s_ref, t_ref, o_ref):
    cnt = jnp.maximum(c_ref[0, :, :1] + c_ref[1, :, :1], 1.0)
    agg = (p_ref[0] + p_ref[1]) / cnt
    h = h_ref[...]
    z = (jnp.dot(agg, wl_ref[...], preferred_element_type=jnp.float32)
         + jnp.dot(h, wr_ref[...], preferred_element_type=jnp.float32))
    o_ref[...] = h + jax.nn.relu(z * s_ref[...] + t_ref[...])


def _tc_layer_head_body(p_ref, c_ref, h_ref, wl_ref, wr_ref, s_ref, t_ref,
                        wc1_ref, bc1_ref, wc2_ref, bc2_ref, o_ref):
    cnt = jnp.maximum(c_ref[0, :, :1] + c_ref[1, :, :1], 1.0)
    agg = (p_ref[0] + p_ref[1]) / cnt
    h = h_ref[...]
    z = (jnp.dot(agg, wl_ref[...], preferred_element_type=jnp.float32)
         + jnp.dot(h, wr_ref[...], preferred_element_type=jnp.float32))
    h2 = h + jax.nn.relu(z * s_ref[...] + t_ref[...])
    y = jax.nn.relu(
        jnp.dot(h2, wc1_ref[...], preferred_element_type=jnp.float32)
        + bc1_ref[...])
    o_ref[...] = (jnp.dot(y, wc2_ref[...], preferred_element_type=jnp.float32)
                  + bc2_ref[...])


def _row(shape):
    return pl.BlockSpec(shape, lambda i: (0,) * len(shape))


def _tc_in(x, w, b):
    return pl.pallas_call(
        _tc_in_body,
        grid=(N // BLK,),
        in_specs=[
            pl.BlockSpec((BLK, H), lambda i: (i, 0)),
            _row((H, H)),
            _row((1, H)),
        ],
        out_specs=pl.BlockSpec((BLK, H), lambda i: (i, 0)),
        out_shape=jax.ShapeDtypeStruct((N, H), jnp.float32),
    )(x, w, b)


def _tc_layer(p, c, h, wl, wr, s, t):
    return pl.pallas_call(
        _tc_layer_body,
        grid=(N // BLK,),
        in_specs=[
            pl.BlockSpec((NC, BLK, H), lambda i: (0, i, 0)),
            pl.BlockSpec((NC, BLK, H), lambda i: (0, i, 0)),
            pl.BlockSpec((BLK, H), lambda i: (i, 0)),
            _row((H, H)), _row((H, H)), _row((1, H)), _row((1, H)),
        ],
        out_specs=pl.BlockSpec((BLK, H), lambda i: (i, 0)),
        out_shape=jax.ShapeDtypeStruct((N, H), jnp.float32),
    )(p, c, h, wl, wr, s, t)


def _tc_layer_head(p, c, h, wl, wr, s, t, wc1, bc1, wc2, bc2):
    return pl.pallas_call(
        _tc_layer_head_body,
        grid=(N // BLK,),
        in_specs=[
            pl.BlockSpec((NC, BLK, H), lambda i: (0, i, 0)),
            pl.BlockSpec((NC, BLK, H), lambda i: (0, i, 0)),
            pl.BlockSpec((BLK, H), lambda i: (i, 0)),
            _row((H, H)), _row((H, H)), _row((1, H)), _row((1, H)),
            _row((H, H // 2)), _row((1, H // 2)),
            _row((H // 2, 1)), _row((1, 1)),
        ],
        out_specs=pl.BlockSpec((BLK, 1), lambda i: (i, 0)),
        out_shape=jax.ShapeDtypeStruct((N, 1), jnp.float32),
    )(p, c, h, wl, wr, s, t, wc1, bc1, wc2, bc2)


def _fold_bn(g, be, rm, rv, bl, eps=1e-5):
    s = g / jnp.sqrt(rv + eps)
    t = be - rm * s + bl * s
    return s.reshape(1, H), t.reshape(1, H)


def kernel(x, edge_index, edge_attr, W_in, b_in, Wl0, bl0, Wr0, g0, be0, rm0,
           rv0, Wl1, bl1, Wr1, g1, be1, rm1, rv1, Wc1, bc1, Wc2, bc2):
    src = edge_index[0]
    dst = edge_index[1]
    npad = E_PAD - E
    pad_src = (jnp.arange(npad, dtype=jnp.int32) * 131) % N
    pad_dst = N + (jnp.arange(npad, dtype=jnp.int32) % (R_PAD - N))
    src_r = jnp.concatenate([src, pad_src]).reshape(NW, CPW, CHUNK)
    dst_r = jnp.concatenate([dst, pad_dst]).reshape(NW, CPW, CHUNK)

    s0, t0 = _fold_bn(g0, be0, rm0, rv0, bl0)
    s1, t1 = _fold_bn(g1, be1, rm1, rv1, bl1)

    h0 = _tc_in(x, W_in, b_in.reshape(1, H))
    p0, cnt = _make_sc_agg(True)(h0, src_r, dst_r)
    h1 = _tc_layer(p0, cnt, h0, Wl0, Wr0, s0, t0)
    (p1,) = _make_sc_agg(False)(h1, src_r, dst_r)
    return _tc_layer_head(p1, cnt, h1, Wl1, Wr1, s1, t1,
                          Wc1, bc1.reshape(1, H // 2), Wc2, bc2.reshape(1, 1))


# speedup vs baseline: 9.5559x; 1.0306x over previous
import functools

import jax
import jax.numpy as jnp
from jax import lax
from jax.experimental import pallas as pl
from jax.experimental.pallas import tpu as pltpu
from jax.experimental.pallas import tpu_sc as plsc

N = 10000
H = 128
E = 320000

NC = 2
NS = 16
NW = NC * NS
CHUNK = 64
CPW = 160
NPH = 2
PPC = CPW // NPH
E_PAD = NW * CPW * CHUNK
R_PAD = 10240
RPW = R_PAD // NS

BLK = 1000


def _fill_rows(rows_v, val):
    v = jnp.full((16,), val, jnp.float32)

    def frow(i, carry):
        for k in range(H // 16):
            rows_v[i, pl.ds(k * 16, 16)] = v
        return carry

    lax.fori_loop(0, CHUNK, frow, 0)


def _zero_acc(rows_v, agg_s, rowbase, sem):
    _fill_rows(rows_v, 0.0)
    nblk = RPW // CHUNK
    for k in range(nblk):
        pltpu.async_copy(rows_v, agg_s.at[pl.ds(rowbase + k * CHUNK, CHUNK)],
                         sem)
    for k in range(nblk):
        pltpu.make_async_copy(rows_v, agg_s.at[pl.ds(rowbase, CHUNK)],
                              sem).wait()


def _flush_acc(b0, b1, agg_s, out_hbm, cid, rowbase, sem0, sem1):
    bufs = (b0, b1)
    sems = (sem0, sem1)
    nblk = RPW // CHUNK
    for k in range(nblk):
        b = k % 2
        r = rowbase + k * CHUNK
        if k >= 2:
            pltpu.make_async_copy(bufs[b], out_hbm.at[cid, pl.ds(0, CHUNK)],
                                  sems[b]).wait()
        pltpu.sync_copy(agg_s.at[pl.ds(r, CHUNK)], bufs[b])
        pltpu.async_copy(bufs[b], out_hbm.at[cid, pl.ds(r, CHUNK)], sems[b])
    for b in range(2):
        pltpu.make_async_copy(bufs[b], out_hbm.at[cid, pl.ds(0, CHUNK)],
                              sems[b]).wait()


def _sc_agg_body(with_counts, *refs):
    if with_counts:
        (h_hbm, src_hbm, dst_hbm,
         p_hbm, cnt_hbm,
         src_v, dst_v, rows0, rows1, agg_s, sem0, sem1) = refs
    else:
        (h_hbm, src_hbm, dst_hbm,
         p_hbm,
         src_v, dst_v, rows0, rows1, agg_s, sem0, sem1) = refs

    cid = lax.axis_index("c")
    sid = lax.axis_index("s")
    wid = sid * NC + cid
    rowbase = sid * RPW

    if with_counts:
        _zero_acc(rows0, agg_s, rowbase, sem0)
        plsc.subcore_barrier()
        _fill_rows(rows0, 1.0)

        def cfire(j, carry):
            pltpu.async_copy(rows0, agg_s.at[dst_v.at[j]], sem0, add=True)
            return carry

        def cdrain(j, carry):
            pltpu.make_async_copy(rows0, agg_s.at[dst_v.at[0]], sem0).wait()
            return carry

        for ph in range(NPH):
            pltpu.sync_copy(dst_hbm.at[wid, pl.ds(ph * PPC, PPC)], dst_v)
            lax.fori_loop(0, PPC, cfire, 0)
            lax.fori_loop(0, PPC, cdrain, 0)
        plsc.subcore_barrier()
        _flush_acc(rows0, rows1, agg_s, cnt_hbm, cid, rowbase, sem0, sem1)
        plsc.subcore_barrier()

    _zero_acc(rows0, agg_s, rowbase, sem0)
    plsc.subcore_barrier()

    bufs = (rows0, rows1)
    sems = (sem0, sem1)

    def pair(g, carry):
        for b in range(2):
            j = 2 * g + b
            pltpu.make_async_copy(h_hbm.at[src_v.at[0]], bufs[b],
                                  sems[b]).wait()
            pltpu.sync_copy(bufs[b], agg_s.at[dst_v.at[j]], add=True)
            pltpu.async_copy(h_hbm.at[src_v.at[j + 2]], bufs[b], sems[b])
        return carry

    for ph in range(NPH):
        pltpu.sync_copy(src_hbm.at[wid, pl.ds(ph * PPC, PPC)], src_v)
        pltpu.sync_copy(dst_hbm.at[wid, pl.ds(ph * PPC, PPC)], dst_v)
        pltpu.async_copy(h_hbm.at[src_v.at[0]], rows0, sem0)
        pltpu.async_copy(h_hbm.at[src_v.at[1]], rows1, sem1)
        lax.fori_loop(0, PPC // 2 - 1, pair, 0)
        for b in range(2):
            pltpu.make_async_copy(h_hbm.at[src_v.at[0]], bufs[b],
                                  sems[b]).wait()
            pltpu.sync_copy(bufs[b], agg_s.at[dst_v.at[PPC - 2 + b]],
                            add=True)

    plsc.subcore_barrier()
    _flush_acc(rows0, rows1, agg_s, p_hbm, cid, rowbase, sem0, sem1)


def _make_sc_agg(with_counts):
    mesh = plsc.VectorSubcoreMesh(core_axis_name="c", subcore_axis_name="s")
    out_type = [jax.ShapeDtypeStruct((NC, R_PAD, H), jnp.float32)]
    scratch = [
        pltpu.VMEM((PPC, CHUNK), jnp.int32),
        pltpu.VMEM((PPC, CHUNK), jnp.int32),
        pltpu.VMEM((CHUNK, H), jnp.float32),
        pltpu.VMEM((CHUNK, H), jnp.float32),
    ]
    if with_counts:
        out_type.append(jax.ShapeDtypeStruct((NC, R_PAD, H), jnp.float32))
    scratch.append(pltpu.VMEM_SHARED((R_PAD, H), jnp.float32))
    scratch.append(pltpu.SemaphoreType.DMA)
    scratch.append(pltpu.SemaphoreType.DMA)
    return pl.kernel(
        functools.partial(_sc_agg_body, with_counts),
        out_type=tuple(out_type),
        mesh=mesh,
        scratch_types=scratch,
        name="sc_segment_sum" + ("_cnt" if with_counts else ""),
    )


def _tc_in_body(x_ref, w_ref, b_ref, o_ref):
    o_ref[...] = jax.nn.relu(
        jnp.dot(x_ref[...], w_ref[...], preferred_element_type=jnp.float32)
        + b_ref[...])


def _tc_layer_body(p_ref, c_ref, h_ref, wl_ref, wr_ref, s_ref, t_ref, o_ref):
    cnt = jnp.maximum(c_ref[0, :, :1] + c_ref[1, :, :1], 1.0)
    agg = (p_ref[0] + p_ref[1]) / cnt
    h = h_ref[...]
    z = (jnp.dot(agg, wl_ref[...], preferred_element_type=jnp.float32)
         + jnp.dot(h, wr_ref[...], preferred_element_type=jnp.float32))
    o_ref[...] = h + jax.nn.relu(z * s_ref[...] + t_ref[...])


def _tc_layer_head_body(p_ref, c_ref, h_ref, wl_ref, wr_ref, s_ref, t_ref,
                        wc1_ref, bc1_ref, wc2_ref, bc2_ref, o_ref):
    cnt = jnp.maximum(c_ref[0, :, :1] + c_ref[1, :, :1], 1.0)
    agg = (p_ref[0] + p_ref[1]) / cnt
    h = h_ref[...]
    z = (jnp.dot(agg, wl_ref[...], preferred_element_type=jnp.float32)
         + jnp.dot(h, wr_ref[...], preferred_element_type=jnp.float32))
    h2 = h + jax.nn.relu(z * s_ref[...] + t_ref[...])
    y = jax.nn.relu(
        jnp.dot(h2, wc1_ref[...], preferred_element_type=jnp.float32)
        + bc1_ref[...])
    o_ref[...] = (jnp.dot(y, wc2_ref[...], preferred_element_type=jnp.float32)
                  + bc2_ref[...])


def _row(shape):
    return pl.BlockSpec(shape, lambda i: (0,) * len(shape))


def _tc_in(x, w, b):
    return pl.pallas_call(
        _tc_in_body,
        grid=(N // BLK,),
        in_specs=[
            pl.BlockSpec((BLK, H), lambda i: (i, 0)),
            _row((H, H)),
            _row((1, H)),
        ],
        out_specs=pl.BlockSpec((BLK, H), lambda i: (i, 0)),
        out_shape=jax.ShapeDtypeStruct((N, H), jnp.float32),
    )(x, w, b)


def _tc_layer(p, c, h, wl, wr, s, t):
    return pl.pallas_call(
        _tc_layer_body,
        grid=(N // BLK,),
        in_specs=[
            pl.BlockSpec((NC, BLK, H), lambda i: (0, i, 0)),
            pl.BlockSpec((NC, BLK, H), lambda i: (0, i, 0)),
            pl.BlockSpec((BLK, H), lambda i: (i, 0)),
            _row((H, H)), _row((H, H)), _row((1, H)), _row((1, H)),
        ],
        out_specs=pl.BlockSpec((BLK, H), lambda i: (i, 0)),
        out_shape=jax.ShapeDtypeStruct((N, H), jnp.float32),
    )(p, c, h, wl, wr, s, t)


def _tc_layer_head(p, c, h, wl, wr, s, t, wc1, bc1, wc2, bc2):
    return pl.pallas_call(
        _tc_layer_head_body,
        grid=(N // BLK,),
        in_specs=[
            pl.BlockSpec((NC, BLK, H), lambda i: (0, i, 0)),
            pl.BlockSpec((NC, BLK, H), lambda i: (0, i, 0)),
            pl.BlockSpec((BLK, H), lambda i: (i, 0)),
            _row((H, H)), _row((H, H)), _row((1, H)), _row((1, H)),
            _row((H, H // 2)), _row((1, H // 2)),
            _row((H // 2, 1)), _row((1, 1)),
        ],
        out_specs=pl.BlockSpec((BLK, 1), lambda i: (i, 0)),
        out_shape=jax.ShapeDtypeStruct((N, 1), jnp.float32),
    )(p, c, h, wl, wr, s, t, wc1, bc1, wc2, bc2)


def _fold_bn(g, be, rm, rv, bl, eps=1e-5):
    s = g / jnp.sqrt(rv + eps)
    t = be - rm * s + bl * s
    return s.reshape(1, H), t.reshape(1, H)


def kernel(x, edge_index, edge_attr, W_in, b_in, Wl0, bl0, Wr0, g0, be0, rm0,
           rv0, Wl1, bl1, Wr1, g1, be1, rm1, rv1, Wc1, bc1, Wc2, bc2):
    src = edge_index[0]
    dst = edge_index[1]
    npad = E_PAD - E
    pad_src = (jnp.arange(npad, dtype=jnp.int32) * 131) % N
    pad_dst = N + (jnp.arange(npad, dtype=jnp.int32) % (R_PAD - N))
    src_r = jnp.concatenate([src, pad_src]).reshape(NW, CPW, CHUNK)
    dst_r = jnp.concatenate([dst, pad_dst]).reshape(NW, CPW, CHUNK)

    s0, t0 = _fold_bn(g0, be0, rm0, rv0, bl0)
    s1, t1 = _fold_bn(g1, be1, rm1, rv1, bl1)

    h0 = _tc_in(x, W_in, b_in.reshape(1, H))
    p0, cnt = _make_sc_agg(True)(h0, src_r, dst_r)
    h1 = _tc_layer(p0, cnt, h0, Wl0, Wr0, s0, t0)
    (p1,) = _make_sc_agg(False)(h1, src_r, dst_r)
    return _tc_layer_head(p1, cnt, h1, Wl1, Wr1, s1, t1,
                          Wc1, bc1.reshape(1, H // 2), Wc2, bc2.reshape(1, 1))


# speedup vs baseline: 9.5746x; 1.0020x over previous
import functools

import jax
import jax.numpy as jnp
from jax import lax
from jax.experimental import pallas as pl
from jax.experimental.pallas import tpu as pltpu
from jax.experimental.pallas import tpu_sc as plsc

N = 10000
H = 128
E = 320000

NC = 2
NS = 16
NW = NC * NS
CHUNK = 64
CPW = 160
NPH = 2
PPC = CPW // NPH
E_PAD = NW * CPW * CHUNK
R_PAD = 10240
RPW = R_PAD // NS

BLK = 1000


def _fill_rows(rows_v, val):
    v = jnp.full((16,), val, jnp.float32)

    def frow(i, carry):
        for k in range(H // 16):
            rows_v[i, pl.ds(k * 16, 16)] = v
        return carry

    lax.fori_loop(0, CHUNK, frow, 0)


def _zero_acc(rows_v, agg_s, rowbase, sem):
    _fill_rows(rows_v, 0.0)
    nblk = RPW // CHUNK
    for k in range(nblk):
        pltpu.async_copy(rows_v, agg_s.at[pl.ds(rowbase + k * CHUNK, CHUNK)],
                         sem)
    for k in range(nblk):
        pltpu.make_async_copy(rows_v, agg_s.at[pl.ds(rowbase, CHUNK)],
                              sem).wait()


def _flush_acc(b0, b1, agg_s, out_hbm, cid, rowbase, sem0, sem1):
    bufs = (b0, b1)
    sems = (sem0, sem1)
    nblk = RPW // CHUNK
    for k in range(nblk):
        b = k % 2
        r = rowbase + k * CHUNK
        if k >= 2:
            pltpu.make_async_copy(bufs[b], out_hbm.at[cid, pl.ds(0, CHUNK)],
                                  sems[b]).wait()
        pltpu.sync_copy(agg_s.at[pl.ds(r, CHUNK)], bufs[b])
        pltpu.async_copy(bufs[b], out_hbm.at[cid, pl.ds(r, CHUNK)], sems[b])
    for b in range(2):
        pltpu.make_async_copy(bufs[b], out_hbm.at[cid, pl.ds(0, CHUNK)],
                              sems[b]).wait()


def _sc_agg_body(with_counts, *refs):
    if with_counts:
        (h_hbm, src_hbm, dst_hbm,
         p_hbm, cnt_hbm,
         src_v, dst_v, rows0, rows1, agg_s, sem0, sem1) = refs
    else:
        (h_hbm, src_hbm, dst_hbm,
         p_hbm,
         src_v, dst_v, rows0, rows1, agg_s, sem0, sem1) = refs

    cid = lax.axis_index("c")
    sid = lax.axis_index("s")
    wid = sid * NC + cid
    rowbase = sid * RPW

    if with_counts:
        _zero_acc(rows0, agg_s, rowbase, sem0)
        plsc.subcore_barrier()
        _fill_rows(rows0, 1.0)

        def cfire(j, carry):
            pltpu.async_copy(rows0, agg_s.at[dst_v.at[j]], sem0, add=True)
            return carry

        def cdrain(j, carry):
            pltpu.make_async_copy(rows0, agg_s.at[dst_v.at[0]], sem0).wait()
            return carry

        for ph in range(NPH):
            pltpu.sync_copy(dst_hbm.at[wid, pl.ds(ph * PPC, PPC)], dst_v)
            lax.fori_loop(0, PPC, cfire, 0)
            lax.fori_loop(0, PPC, cdrain, 0)
        plsc.subcore_barrier()
        _flush_acc(rows0, rows1, agg_s, cnt_hbm, cid, rowbase, sem0, sem1)
        plsc.subcore_barrier()

    _zero_acc(rows0, agg_s, rowbase, sem0)
    plsc.subcore_barrier()

    bufs = (rows0, rows1)
    sems = (sem0, sem1)

    def pair(g, carry):
        for b in range(2):
            j = 2 * g + b
            pltpu.make_async_copy(h_hbm.at[src_v.at[0]], bufs[b],
                                  sems[b]).wait()
            pltpu.sync_copy(bufs[b], agg_s.at[dst_v.at[j]], add=True)
            pltpu.async_copy(h_hbm.at[src_v.at[j + 2]], bufs[b], sems[b])
        return carry

    for ph in range(NPH):
        pltpu.sync_copy(src_hbm.at[wid, pl.ds(ph * PPC, PPC)], src_v)
        pltpu.sync_copy(dst_hbm.at[wid, pl.ds(ph * PPC, PPC)], dst_v)
        pltpu.async_copy(h_hbm.at[src_v.at[0]], rows0, sem0)
        pltpu.async_copy(h_hbm.at[src_v.at[1]], rows1, sem1)
        lax.fori_loop(0, PPC // 2 - 1, pair, 0)
        for b in range(2):
            pltpu.make_async_copy(h_hbm.at[src_v.at[0]], bufs[b],
                                  sems[b]).wait()
            pltpu.sync_copy(bufs[b], agg_s.at[dst_v.at[PPC - 2 + b]],
                            add=True)

    plsc.subcore_barrier()
    _flush_acc(rows0, rows1, agg_s, p_hbm, cid, rowbase, sem0, sem1)


def _make_sc_agg(with_counts):
    mesh = plsc.VectorSubcoreMesh(core_axis_name="c", subcore_axis_name="s")
    out_type = [jax.ShapeDtypeStruct((NC, R_PAD, H), jnp.float32)]
    if with_counts:
        out_type.append(jax.ShapeDtypeStruct((NC, R_PAD, H), jnp.float32))
    return pl.kernel(
        functools.partial(_sc_agg_body, with_counts),
        out_type=tuple(out_type),
        mesh=mesh,
        scratch_types=[
            pltpu.VMEM((PPC, CHUNK), jnp.int32),
            pltpu.VMEM((PPC, CHUNK), jnp.int32),
            pltpu.VMEM((CHUNK, H), jnp.float32),
            pltpu.VMEM((CHUNK, H), jnp.float32),
            pltpu.VMEM_SHARED((R_PAD, H), jnp.float32),
            pltpu.SemaphoreType.DMA,
            pltpu.SemaphoreType.DMA,
        ],
        name="sc_segment_sum" + ("_cnt" if with_counts else ""),
    )


def _tc_in_body(x_ref, w_ref, b_ref, o_ref):
    o_ref[...] = jax.nn.relu(
        jnp.dot(x_ref[...], w_ref[...], preferred_element_type=jnp.float32)
        + b_ref[...])


def _tc_layer_body(p_ref, c_ref, h_ref, wl_ref, wr_ref, s_ref, t_ref,
                   o_ref, inv_ref):
    inv = 1.0 / jnp.maximum(c_ref[0, :, :1] + c_ref[1, :, :1], 1.0)
    agg = (p_ref[0] + p_ref[1]) * inv
    h = h_ref[...]
    z = (jnp.dot(agg, wl_ref[...], preferred_element_type=jnp.float32)
         + jnp.dot(h, wr_ref[...], preferred_element_type=jnp.float32))
    o_ref[...] = h + jax.nn.relu(z * s_ref[...] + t_ref[...])
    inv_ref[...] = inv


def _tc_head_body(p_ref, c_ref, h_ref, wl_ref, wr_ref, s_ref, t_ref,
                  wc1_ref, bc1_ref, wc2_ref, bc2_ref, o_ref):
    agg = (p_ref[0] + p_ref[1]) * c_ref[...]
    h = h_ref[...]
    z = (jnp.dot(agg, wl_ref[...], preferred_element_type=jnp.float32)
         + jnp.dot(h, wr_ref[...], preferred_element_type=jnp.float32))
    h2 = h + jax.nn.relu(z * s_ref[...] + t_ref[...])
    y = jax.nn.relu(
        jnp.dot(h2, wc1_ref[...], preferred_element_type=jnp.float32)
        + bc1_ref[...])
    o_ref[...] = (jnp.dot(y, wc2_ref[...], preferred_element_type=jnp.float32)
                  + bc2_ref[...])


def _row(shape):
    return pl.BlockSpec(shape, lambda i: (0,) * len(shape))


def _tc_in(x, w, b):
    return pl.pallas_call(
        _tc_in_body,
        grid=(N // BLK,),
        in_specs=[
            pl.BlockSpec((BLK, H), lambda i: (i, 0)),
            _row((H, H)),
            _row((1, H)),
        ],
        out_specs=pl.BlockSpec((BLK, H), lambda i: (i, 0)),
        out_shape=jax.ShapeDtypeStruct((N, H), jnp.float32),
    )(x, w, b)


def _tc_layer(p, c, h, wl, wr, s, t):
    return pl.pallas_call(
        _tc_layer_body,
        grid=(N // BLK,),
        in_specs=[
            pl.BlockSpec((NC, BLK, H), lambda i: (0, i, 0)),
            pl.BlockSpec((NC, BLK, H), lambda i: (0, i, 0)),
            pl.BlockSpec((BLK, H), lambda i: (i, 0)),
            _row((H, H)), _row((H, H)), _row((1, H)), _row((1, H)),
        ],
        out_specs=[
            pl.BlockSpec((BLK, H), lambda i: (i, 0)),
            pl.BlockSpec((BLK, 1), lambda i: (i, 0)),
        ],
        out_shape=[
            jax.ShapeDtypeStruct((N, H), jnp.float32),
            jax.ShapeDtypeStruct((N, 1), jnp.float32),
        ],
    )(p, c, h, wl, wr, s, t)


def _tc_head(p, c, h, wl, wr, s, t, wc1, bc1, wc2, bc2):
    return pl.pallas_call(
        _tc_head_body,
        grid=(N // BLK,),
        in_specs=[
            pl.BlockSpec((NC, BLK, H), lambda i: (0, i, 0)),
            pl.BlockSpec((BLK, 1), lambda i: (i, 0)),
            pl.BlockSpec((BLK, H), lambda i: (i, 0)),
            _row((H, H)), _row((H, H)), _row((1, H)), _row((1, H)),
            _row((H, H // 2)), _row((1, H // 2)),
            _row((H // 2, 1)), _row((1, 1)),
        ],
        out_specs=pl.BlockSpec((BLK, 1), lambda i: (i, 0)),
        out_shape=jax.ShapeDtypeStruct((N, 1), jnp.float32),
    )(p, c, h, wl, wr, s, t, wc1, bc1, wc2, bc2)


def _fold_bn(g, be, rm, rv, bl, eps=1e-5):
    s = g / jnp.sqrt(rv + eps)
    t = be - rm * s + bl * s
    return s.reshape(1, H), t.reshape(1, H)


def kernel(x, edge_index, edge_attr, W_in, b_in, Wl0, bl0, Wr0, g0, be0, rm0,
           rv0, Wl1, bl1, Wr1, g1, be1, rm1, rv1, Wc1, bc1, Wc2, bc2):
    src = edge_index[0]
    dst = edge_index[1]
    npad = E_PAD - E
    pad_src = (jnp.arange(npad, dtype=jnp.int32) * 131) % N
    pad_dst = N + (jnp.arange(npad, dtype=jnp.int32) % (R_PAD - N))
    src_r = jnp.concatenate([src, pad_src]).reshape(NW, CPW, CHUNK)
    dst_r = jnp.concatenate([dst, pad_dst]).reshape(NW, CPW, CHUNK)

    s0, t0 = _fold_bn(g0, be0, rm0, rv0, bl0)
    s1, t1 = _fold_bn(g1, be1, rm1, rv1, bl1)

    h0 = _tc_in(x, W_in, b_in.reshape(1, H))
    p0, cnt = _make_sc_agg(True)(h0, src_r, dst_r)
    h1, inv = _tc_layer(p0, cnt, h0, Wl0, Wr0, s0, t0)
    (p1,) = _make_sc_agg(False)(h1, src_r, dst_r)
    return _tc_head(p1, inv, h1, Wl1, Wr1, s1, t1,
                    Wc1, bc1.reshape(1, H // 2), Wc2, bc2.reshape(1, 1))


# speedup vs baseline: 11.1376x; 1.1632x over previous
import functools

import jax
import jax.numpy as jnp
from jax import lax
from jax.experimental import pallas as pl
from jax.experimental.pallas import tpu as pltpu
from jax.experimental.pallas import tpu_sc as plsc

N = 10000
H = 128
E = 320000

NC = 2
NS = 16
NW = NC * NS
CHUNK = 64
CPW = 160
NPH = 2
PPC = CPW // NPH
E_PAD = NW * CPW * CHUNK
R_PAD = 10240
RPW = R_PAD // NS

BLK = 1000


def _fill_rows(rows_v, val):
    v = jnp.full((16,), val, jnp.float32)

    def frow(i, carry):
        for k in range(H // 16):
            rows_v[i, pl.ds(k * 16, 16)] = v
        return carry

    lax.fori_loop(0, CHUNK, frow, 0)


def _zero_acc(rows_v, agg_s, rowbase, sem):
    _fill_rows(rows_v, 0.0)
    nblk = RPW // CHUNK
    for k in range(nblk):
        pltpu.async_copy(rows_v, agg_s.at[pl.ds(rowbase + k * CHUNK, CHUNK)],
                         sem)
    for k in range(nblk):
        pltpu.make_async_copy(rows_v, agg_s.at[pl.ds(rowbase, CHUNK)],
                              sem).wait()


def _flush_acc(b0, b1, agg_s, out_hbm, cid, rowbase, sem0, sem1):
    bufs = (b0, b1)
    sems = (sem0, sem1)
    nblk = RPW // CHUNK
    for k in range(nblk):
        b = k % 2
        r = rowbase + k * CHUNK
        if k >= 2:
            pltpu.make_async_copy(bufs[b], out_hbm.at[cid, pl.ds(0, CHUNK)],
                                  sems[b]).wait()
        pltpu.sync_copy(agg_s.at[pl.ds(r, CHUNK)], bufs[b])
        pltpu.async_copy(bufs[b], out_hbm.at[cid, pl.ds(r, CHUNK)], sems[b])
    for b in range(2):
        pltpu.make_async_copy(bufs[b], out_hbm.at[cid, pl.ds(0, CHUNK)],
                              sems[b]).wait()


def _sc_agg_body(with_counts, *refs):
    if with_counts:
        (h_hbm, src_hbm, dst_hbm,
         p_hbm, cnt_hbm,
         src_v, dst_v, rows0, rows1, rows2, agg_s, sem0, sem1, sem2) = refs
    else:
        (h_hbm, src_hbm, dst_hbm,
         p_hbm,
         src_v, dst_v, rows0, rows1, rows2, agg_s, sem0, sem1, sem2) = refs

    cid = lax.axis_index("c")
    sid = lax.axis_index("s")
    wid = sid * NC + cid
    rowbase = sid * RPW

    if with_counts:
        _zero_acc(rows0, agg_s, rowbase, sem0)
        plsc.subcore_barrier()
        _fill_rows(rows0, 1.0)

        def cfire(j, carry):
            pltpu.async_copy(rows0, agg_s.at[dst_v.at[j]], sem0, add=True)
            return carry

        def cdrain(j, carry):
            pltpu.make_async_copy(rows0, agg_s.at[dst_v.at[0]], sem0).wait()
            return carry

        for ph in range(NPH):
            pltpu.sync_copy(dst_hbm.at[wid, pl.ds(ph * PPC, PPC)], dst_v)
            lax.fori_loop(0, PPC, cfire, 0)
            lax.fori_loop(0, PPC, cdrain, 0)
        plsc.subcore_barrier()
        _flush_acc(rows0, rows1, agg_s, cnt_hbm, cid, rowbase, sem0, sem1)
        plsc.subcore_barrier()

    _zero_acc(rows0, agg_s, rowbase, sem0)
    plsc.subcore_barrier()

    bufs = (rows0, rows1, rows2)
    sems = (sem0, sem1, sem2)
    ntrip = PPC // 3 - 1

    def trip(g, carry):
        for b in range(3):
            j = 3 * g + b
            pltpu.make_async_copy(h_hbm.at[src_v.at[0]], bufs[b],
                                  sems[b]).wait()
            pltpu.sync_copy(bufs[b], agg_s.at[dst_v.at[j]], add=True)
            pltpu.async_copy(h_hbm.at[src_v.at[j + 3]], bufs[b], sems[b])
        return carry

    for ph in range(NPH):
        pltpu.sync_copy(src_hbm.at[wid, pl.ds(ph * PPC, PPC)], src_v)
        pltpu.sync_copy(dst_hbm.at[wid, pl.ds(ph * PPC, PPC)], dst_v)
        for b in range(3):
            pltpu.async_copy(h_hbm.at[src_v.at[b]], bufs[b], sems[b])
        lax.fori_loop(0, ntrip, trip, 0)
        for j in range(3 * ntrip, PPC):
            b = j % 3
            pltpu.make_async_copy(h_hbm.at[src_v.at[0]], bufs[b],
                                  sems[b]).wait()
            pltpu.sync_copy(bufs[b], agg_s.at[dst_v.at[j]], add=True)
            if j + 3 < PPC:
                pltpu.async_copy(h_hbm.at[src_v.at[j + 3]], bufs[b], sems[b])

    plsc.subcore_barrier()
    _flush_acc(rows0, rows1, agg_s, p_hbm, cid, rowbase, sem0, sem1)


def _make_sc_agg(with_counts):
    mesh = plsc.VectorSubcoreMesh(core_axis_name="c", subcore_axis_name="s")
    out_type = [jax.ShapeDtypeStruct((NC, R_PAD, H), jnp.float32)]
    if with_counts:
        out_type.append(jax.ShapeDtypeStruct((NC, R_PAD, H), jnp.float32))
    return pl.kernel(
        functools.partial(_sc_agg_body, with_counts),
        out_type=tuple(out_type),
        mesh=mesh,
        scratch_types=[
            pltpu.VMEM((PPC, CHUNK), jnp.int32),
            pltpu.VMEM((PPC, CHUNK), jnp.int32),
            pltpu.VMEM((CHUNK, H), jnp.float32),
            pltpu.VMEM((CHUNK, H), jnp.float32),
            pltpu.VMEM((CHUNK, H), jnp.float32),
            pltpu.VMEM_SHARED((R_PAD, H), jnp.float32),
            pltpu.SemaphoreType.DMA,
            pltpu.SemaphoreType.DMA,
            pltpu.SemaphoreType.DMA,
        ],
        name="sc_segment_sum" + ("_cnt" if with_counts else ""),
    )


def _tc_in_body(x_ref, w_ref, b_ref, o_ref):
    o_ref[...] = jax.nn.relu(
        jnp.dot(x_ref[...], w_ref[...], preferred_element_type=jnp.float32)
        + b_ref[...])


def _tc_layer_body(p_ref, c_ref, h_ref, wl_ref, wr_ref, s_ref, t_ref,
                   o_ref, inv_ref):
    inv = 1.0 / jnp.maximum(c_ref[0, :, :1] + c_ref[1, :, :1], 1.0)
    agg = (p_ref[0] + p_ref[1]) * inv
    h = h_ref[...]
    z = (jnp.dot(agg, wl_ref[...], preferred_element_type=jnp.float32)
         + jnp.dot(h, wr_ref[...], preferred_element_type=jnp.float32))
    o_ref[...] = h + jax.nn.relu(z * s_ref[...] + t_ref[...])
    inv_ref[...] = inv


def _tc_head_body(p_ref, c_ref, h_ref, wl_ref, wr_ref, s_ref, t_ref,
                  wc1_ref, bc1_ref, wc2_ref, bc2_ref, o_ref):
    agg = (p_ref[0] + p_ref[1]) * c_ref[...]
    h = h_ref[...]
    z = (jnp.dot(agg, wl_ref[...], preferred_element_type=jnp.float32)
         + jnp.dot(h, wr_ref[...], preferred_element_type=jnp.float32))
    h2 = h + jax.nn.relu(z * s_ref[...] + t_ref[...])
    y = jax.nn.relu(
        jnp.dot(h2, wc1_ref[...], preferred_element_type=jnp.float32)
        + bc1_ref[...])
    o_ref[...] = (jnp.dot(y, wc2_ref[...], preferred_element_type=jnp.float32)
                  + bc2_ref[...])


def _row(shape):
    return pl.BlockSpec(shape, lambda i: (0,) * len(shape))


def _tc_in(x, w, b):
    return pl.pallas_call(
        _tc_in_body,
        grid=(N // BLK,),
        in_specs=[
            pl.BlockSpec((BLK, H), lambda i: (i, 0)),
            _row((H, H)),
            _row((1, H)),
        ],
        out_specs=pl.BlockSpec((BLK, H), lambda i: (i, 0)),
        out_shape=jax.ShapeDtypeStruct((N, H), jnp.float32),
    )(x, w, b)


def _tc_layer(p, c, h, wl, wr, s, t):
    return pl.pallas_call(
        _tc_layer_body,
        grid=(N // BLK,),
        in_specs=[
            pl.BlockSpec((NC, BLK, H), lambda i: (0, i, 0)),
            pl.BlockSpec((NC, BLK, H), lambda i: (0, i, 0)),
            pl.BlockSpec((BLK, H), lambda i: (i, 0)),
            _row((H, H)), _row((H, H)), _row((1, H)), _row((1, H)),
        ],
        out_specs=[
            pl.BlockSpec((BLK, H), lambda i: (i, 0)),
            pl.BlockSpec((BLK, 1), lambda i: (i, 0)),
        ],
        out_shape=[
            jax.ShapeDtypeStruct((N, H), jnp.float32),
            jax.ShapeDtypeStruct((N, 1), jnp.float32),
        ],
    )(p, c, h, wl, wr, s, t)


def _tc_head(p, c, h, wl, wr, s, t, wc1, bc1, wc2, bc2):
    return pl.pallas_call(
        _tc_head_body,
        grid=(N // BLK,),
        in_specs=[
            pl.BlockSpec((NC, BLK, H), lambda i: (0, i, 0)),
            pl.BlockSpec((BLK, 1), lambda i: (i, 0)),
            pl.BlockSpec((BLK, H), lambda i: (i, 0)),
            _row((H, H)), _row((H, H)), _row((1, H)), _row((1, H)),
            _row((H, H // 2)), _row((1, H // 2)),
            _row((H // 2, 1)), _row((1, 1)),
        ],
        out_specs=pl.BlockSpec((BLK, 1), lambda i: (i, 0)),
        out_shape=jax.ShapeDtypeStruct((N, 1), jnp.float32),
    )(p, c, h, wl, wr, s, t, wc1, bc1, wc2, bc2)


def _fold_bn(g, be, rm, rv, bl, eps=1e-5):
    s = g / jnp.sqrt(rv + eps)
    t = be - rm * s + bl * s
    return s.reshape(1, H), t.reshape(1, H)


def kernel(x, edge_index, edge_attr, W_in, b_in, Wl0, bl0, Wr0, g0, be0, rm0,
           rv0, Wl1, bl1, Wr1, g1, be1, rm1, rv1, Wc1, bc1, Wc2, bc2):
    src = edge_index[0]
    dst = edge_index[1]
    npad = E_PAD - E
    pad_src = (jnp.arange(npad, dtype=jnp.int32) * 131) % N
    pad_dst = N + (jnp.arange(npad, dtype=jnp.int32) % (R_PAD - N))
    src_r = jnp.concatenate([src, pad_src]).reshape(NW, CPW, CHUNK)
    dst_r = jnp.concatenate([dst, pad_dst]).reshape(NW, CPW, CHUNK)

    s0, t0 = _fold_bn(g0, be0, rm0, rv0, bl0)
    s1, t1 = _fold_bn(g1, be1, rm1, rv1, bl1)

    h0 = _tc_in(x, W_in, b_in.reshape(1, H))
    p0, cnt = _make_sc_agg(True)(h0, src_r, dst_r)
    h1, inv = _tc_layer(p0, cnt, h0, Wl0, Wr0, s0, t0)
    (p1,) = _make_sc_agg(False)(h1, src_r, dst_r)
    return _tc_head(p1, inv, h1, Wl1, Wr1, s1, t1,
                    Wc1, bc1.reshape(1, H // 2), Wc2, bc2.reshape(1, 1))
